# baseline (speedup 1.0000x reference)
import functools

import jax
import jax.numpy as jnp
from jax.experimental import pallas as pl

_SCALING = 8.0


def _router_body(x_ref, wr_ref, bias_ref, coeff_ref):
    logits = jax.lax.dot_general(
        x_ref[...], wr_ref[...],
        dimension_numbers=(((1,), (1,)), ((), ())),
        preferred_element_type=jnp.float32,
    ) + bias_ref[...]
    probs = jax.nn.sigmoid(logits)
    t, e128 = probs.shape
    col = jax.lax.broadcasted_iota(jnp.int32, (t, e128), 1)
    valid = col < 15
    probs = jnp.where(valid, probs, -1.0)
    m1 = jnp.max(probs, axis=1, keepdims=True)
    i1 = jnp.min(jnp.where(probs == m1, col, e128), axis=1, keepdims=True)
    oh1 = col == i1
    probs2 = jnp.where(oh1, -2.0, probs)
    m2 = jnp.max(probs2, axis=1, keepdims=True)
    i2 = jnp.min(jnp.where(probs2 == m2, col, e128), axis=1, keepdims=True)
    oh2 = col == i2
    denom = m1 + m2
    coeff = jnp.where(oh1, m1 / denom, 0.0) + jnp.where(oh2, m2 / denom, 0.0)
    coeff = jnp.where(col == 15, 1.0, coeff)
    coeff_ref[...] = coeff


def _experts_body(x_ref, wg_ref, wu_ref, wd_ref, coeff_ref, out_ref):
    e = pl.program_id(0)
    xb = x_ref[...]
    g = jax.lax.dot_general(
        xb, wg_ref[0],
        dimension_numbers=(((1,), (1,)), ((), ())),
        preferred_element_type=jnp.float32,
    )
    u = jax.lax.dot_general(
        xb, wu_ref[0],
        dimension_numbers=(((1,), (1,)), ((), ())),
        preferred_element_type=jnp.float32,
    )
    inter = (g * jax.nn.sigmoid(g)) * u * (1.0 / _SCALING)
    eo = jax.lax.dot_general(
        inter.astype(jnp.bfloat16), wd_ref[0],
        dimension_numbers=(((1,), (1,)), ((), ())),
        preferred_element_type=jnp.float32,
    )
    t = eo.shape[0]
    col = jax.lax.broadcasted_iota(jnp.int32, (t, 128), 1)
    c_e = jnp.sum(jnp.where(col == e, coeff_ref[...], 0.0), axis=1,
                  keepdims=True)
    contrib = eo * c_e

    @pl.when(e == 0)
    def _init():
        out_ref[...] = contrib

    @pl.when(e > 0)
    def _acc():
        out_ref[...] += contrib


@jax.jit
def kernel(x, Wg_s, Wu_s, Wd_s, Wg, Wu, Wd, Wr, routing_bias):
    b, s, h = x.shape
    t = b * s
    er, i, _ = Wg.shape
    ea = er + 1
    xf = x.reshape(t, h)

    wr_pad = jnp.zeros((128, h), dtype=jnp.float32).at[:er].set(Wr)
    bias_pad = jnp.zeros((1, 128), dtype=jnp.float32).at[0, :er].set(routing_bias)

    coeff = pl.pallas_call(
        _router_body,
        out_shape=jax.ShapeDtypeStruct((t, 128), jnp.float32),
    )(xf, wr_pad, bias_pad)

    wg_all = jnp.concatenate([Wg, Wg_s[None]], axis=0).astype(jnp.bfloat16)
    wu_all = jnp.concatenate([Wu, Wu_s[None]], axis=0).astype(jnp.bfloat16)
    wd_all = jnp.concatenate([Wd, Wd_s[None]], axis=0).astype(jnp.bfloat16)
    x_bf = xf.astype(jnp.bfloat16)

    out = pl.pallas_call(
        _experts_body,
        grid=(ea,),
        in_specs=[
            pl.BlockSpec((t, h), lambda e: (0, 0)),
            pl.BlockSpec((1, i, h), lambda e: (e, 0, 0)),
            pl.BlockSpec((1, i, h), lambda e: (e, 0, 0)),
            pl.BlockSpec((1, h, i), lambda e: (e, 0, 0)),
            pl.BlockSpec((t, 128), lambda e: (0, 0)),
        ],
        out_specs=pl.BlockSpec((t, h), lambda e: (0, 0)),
        out_shape=jax.ShapeDtypeStruct((t, h), jnp.float32),
    )(x_bf, wg_all, wu_all, wd_all, coeff)

    return out.reshape(b, s, h)
